# Initial kernel scaffold; baseline (speedup 1.0000x reference)
import jax
import jax.numpy as jnp
from jax.experimental import pallas as pl


def kernel(condition, centers_eluent, gamma_eluent, W_eluent, b_eluent, centers_grain, gamma_grain, W_grain, b_grain, emb_silica, emb_replace):
    raise NotImplementedError("write your pallas kernel here")


# speedup vs baseline: 5.3015x; 5.3015x over previous
import functools

import jax
import jax.numpy as jnp
from jax.experimental import pallas as pl


def _fused_kernel(cond_ref, ctr_ref, gam_ref, w_ref, bias_ref, out_ref,
                  *, n_e, n_g, n_s):
    bB = cond_ref.shape[0]
    K = ctr_ref.shape[1]
    lane = jax.lax.broadcasted_iota(jnp.int32, (bB, K), 1)
    x1 = cond_ref[:, 1:2]
    x3 = cond_ref[:, 3:4]
    x0 = cond_ref[:, 0:1]
    x2 = cond_ref[:, 2:3]
    e1 = n_e
    e2 = n_e + n_g
    e3 = n_e + n_g + n_s
    x = jnp.where(lane < e1, x1,
                  jnp.where(lane < e2, x3,
                            jnp.where(lane < e3, x0, x2)))
    ctr = ctr_ref[0:1, :]
    gam = gam_ref[0:1, :]
    d = x - ctr
    rbf_val = jnp.exp(-gam * d * d)
    oh_val = (jnp.floor(x) == ctr).astype(jnp.float32)
    feats = jnp.where(lane < e2, rbf_val, oh_val)
    acc = jax.lax.dot_general(
        feats, w_ref[...],
        dimension_numbers=(((1,), (0,)), ((), ())),
        preferred_element_type=jnp.float32,
    )
    out_ref[...] = acc + bias_ref[0:1, :]


def kernel(condition, centers_eluent, gamma_eluent, W_eluent, b_eluent,
           centers_grain, gamma_grain, W_grain, b_grain,
           emb_silica, emb_replace):
    B = condition.shape[0]
    D = W_eluent.shape[1]
    n_e = centers_eluent.shape[0]
    n_g = centers_grain.shape[0]
    n_s = emb_silica.shape[0]
    n_r = emb_replace.shape[0]
    K = n_e + n_g + n_s + n_r

    f32 = jnp.float32
    W_cat = jnp.concatenate([W_eluent, W_grain, emb_silica, emb_replace],
                            axis=0).astype(f32)
    centers_cat = jnp.concatenate([
        centers_eluent.astype(f32),
        centers_grain.astype(f32),
        jnp.arange(n_s, dtype=f32),
        jnp.arange(n_r, dtype=f32),
    ]).reshape(1, K)
    gamma_row = jnp.concatenate([
        jnp.broadcast_to(gamma_eluent.astype(f32).reshape(1), (n_e,)),
        jnp.broadcast_to(gamma_grain.astype(f32).reshape(1), (n_g,)),
        jnp.zeros((n_s + n_r,), f32),
    ]).reshape(1, K)
    bias = (b_eluent + b_grain).astype(f32).reshape(1, D)

    bB = 2048
    grid = (B // bB,)

    out = pl.pallas_call(
        functools.partial(_fused_kernel, n_e=n_e, n_g=n_g, n_s=n_s),
        grid=grid,
        in_specs=[
            pl.BlockSpec((bB, 4), lambda i: (i, 0)),
            pl.BlockSpec((1, K), lambda i: (0, 0)),
            pl.BlockSpec((1, K), lambda i: (0, 0)),
            pl.BlockSpec((K, D), lambda i: (0, 0)),
            pl.BlockSpec((1, D), lambda i: (0, 0)),
        ],
        out_specs=pl.BlockSpec((bB, D), lambda i: (i, 0)),
        out_shape=jax.ShapeDtypeStruct((B, D), f32),
    )(condition.astype(f32), centers_cat, gamma_row, W_cat, bias)
    return out
